# baseline (speedup 1.0000x reference)
import functools

import jax
import jax.numpy as jnp
from jax import lax
from jax.experimental import pallas as pl
from jax.experimental.pallas import tpu as pltpu
from jax.experimental.pallas import tpu_sc as plsc

E = 8
K = 2
D = 768
F = 2048
T = 2048

BT = 256
P = K * T + E * BT
N_TILES = P // BT

NC = 2
NS = 16
NW = NC * NS

ROWS_W = P // NW
GCHUNK = ROWS_W // 3
NCHUNK = ROWS_W // GCHUNK
TOK_W = T // NW


def _router_body(x_ref, gw_ref, out_ref):
    x = x_ref[...]
    logits = lax.dot_general(x, gw_ref[...],
                             (((1,), (1,)), ((), ())),
                             preferred_element_type=jnp.float32)
    m = jnp.max(logits, axis=1, keepdims=True)
    ex = jnp.exp(logits - m)
    p = ex / jnp.sum(ex, axis=1, keepdims=True)
    col = lax.broadcasted_iota(jnp.int32, p.shape, 1)
    a1 = jnp.argmax(p, axis=1)[:, None]
    m1 = col == a1
    a2 = jnp.argmax(jnp.where(m1, -1.0, p), axis=1)[:, None]
    m2 = col == a2
    out_ref[...] = jnp.where(m1 | m2, p, 0.0)


def _router(hs2d, gate_w):
    bt = 256
    return pl.pallas_call(
        _router_body,
        grid=(T // bt,),
        in_specs=[
            pl.BlockSpec((bt, D), lambda i: (i, 0)),
            pl.BlockSpec((E, D), lambda i: (0, 0)),
        ],
        out_specs=pl.BlockSpec((bt, E), lambda i: (i, 0)),
        out_shape=jax.ShapeDtypeStruct((T, E), jnp.float32),
    )(hs2d, gate_w)


def _dispatch(combine):
    rw_top, sel = lax.top_k(combine, K)
    e_flat = sel.reshape(-1).astype(jnp.int32)
    w_flat = rw_top.reshape(-1)
    onehot = (e_flat[:, None] == jnp.arange(E, dtype=jnp.int32)[None, :])
    counts = jnp.sum(onehot.astype(jnp.int32), axis=0)
    rank = jnp.sum((jnp.cumsum(onehot.astype(jnp.int32), axis=0) - 1)
                   * onehot.astype(jnp.int32), axis=1)
    pad_cnt = ((counts + BT - 1) // BT) * BT
    ends = jnp.cumsum(pad_cnt)
    pad_off = ends - pad_cnt
    pos = pad_off[e_flat] + rank

    tok = (jnp.arange(K * T, dtype=jnp.int32) // K)
    token_map = jnp.zeros((P,), jnp.int32).at[pos].set(tok)
    w_col = jnp.zeros((P,), jnp.float32).at[pos].set(w_flat).reshape(P, 1)

    total_pad = ends[-1]
    n_active = total_pad // BT
    row_start = jnp.arange(N_TILES, dtype=jnp.int32) * BT
    owner_raw = jnp.sum((row_start[:, None] >= ends[None, :]).astype(jnp.int32),
                        axis=1)
    tile_valid = (row_start < total_pad).astype(jnp.int32)
    last_owner = jnp.minimum(owner_raw, E - 1)[jnp.maximum(n_active - 1, 0)]
    tile_expert = jnp.where(tile_valid == 1,
                            jnp.minimum(owner_raw, E - 1), last_owner)
    tile_expert = tile_expert.astype(jnp.int32)

    pos_a = pos[0::K]
    pos_b = pos[1::K]
    return token_map, w_col, tile_expert, tile_valid, pos_a, pos_b


def _sc_gather_body(hs_hbm, tm_hbm, xs_hbm, idx_v, buf_a, buf_b, sem_a, sem_b):
    wid = lax.axis_index("s") * NC + lax.axis_index("c")
    base = wid * ROWS_W
    pltpu.sync_copy(tm_hbm.at[wid], idx_v)
    g0 = pltpu.async_copy(hs_hbm.at[idx_v.at[0]], buf_a, sem_a)
    g1 = pltpu.async_copy(hs_hbm.at[idx_v.at[1]], buf_b, sem_b)
    g0.wait()
    w0 = pltpu.async_copy(buf_a, xs_hbm.at[pl.ds(base, GCHUNK)], sem_a)
    g1.wait()
    w1 = pltpu.async_copy(buf_b, xs_hbm.at[pl.ds(base + GCHUNK, GCHUNK)], sem_b)
    w0.wait()
    g2 = pltpu.async_copy(hs_hbm.at[idx_v.at[2]], buf_a, sem_a)
    g2.wait()
    w2 = pltpu.async_copy(buf_a, xs_hbm.at[pl.ds(base + 2 * GCHUNK, GCHUNK)],
                          sem_a)
    w1.wait()
    w2.wait()


def _sc_gather(hs2d, token_map):
    tm = token_map.reshape(NW, NCHUNK, GCHUNK)
    mesh = plsc.VectorSubcoreMesh(core_axis_name="c", subcore_axis_name="s")
    run = functools.partial(
        pl.kernel, mesh=mesh,
        out_type=jax.ShapeDtypeStruct((P, D), jnp.float32),
        scratch_types=[
            pltpu.VMEM((NCHUNK, GCHUNK), jnp.int32),
            pltpu.VMEM((GCHUNK, D), jnp.float32),
            pltpu.VMEM((GCHUNK, D), jnp.float32),
            pltpu.SemaphoreType.DMA,
            pltpu.SemaphoreType.DMA,
        ],
    )(_sc_gather_body)
    return run(hs2d, tm)


def _gmm_body(te_ref, tv_ref, x_ref, g_ref, u_ref, d_ref, w_ref, y_ref):
    i = pl.program_id(0)

    @pl.when(tv_ref[i] != 0)
    def _():
        x = x_ref[...]
        g = lax.dot_general(x, g_ref[0], (((1,), (1,)), ((), ())),
                            preferred_element_type=jnp.float32)
        u = lax.dot_general(x, u_ref[0], (((1,), (1,)), ((), ())),
                            preferred_element_type=jnp.float32)
        act = g * jax.nn.sigmoid(g) * u
        y = lax.dot_general(act, d_ref[0], (((1,), (1,)), ((), ())),
                            preferred_element_type=jnp.float32)
        y_ref[...] = y * w_ref[...]


def _gmm(tile_expert, tile_valid, xs, w_col, gate_proj, up_proj, down_proj):
    grid_spec = pltpu.PrefetchScalarGridSpec(
        num_scalar_prefetch=2,
        grid=(N_TILES,),
        in_specs=[
            pl.BlockSpec((BT, D), lambda i, te, tv: (i, 0)),
            pl.BlockSpec((1, F, D), lambda i, te, tv: (te[i], 0, 0)),
            pl.BlockSpec((1, F, D), lambda i, te, tv: (te[i], 0, 0)),
            pl.BlockSpec((1, D, F), lambda i, te, tv: (te[i], 0, 0)),
            pl.BlockSpec((BT, 1), lambda i, te, tv: (i, 0)),
        ],
        out_specs=pl.BlockSpec((BT, D), lambda i, te, tv: (i, 0)),
    )
    return pl.pallas_call(
        _gmm_body,
        grid_spec=grid_spec,
        out_shape=jax.ShapeDtypeStruct((P, D), jnp.float32),
        compiler_params=pltpu.CompilerParams(
            dimension_semantics=("arbitrary",),
        ),
    )(tile_expert, tile_valid, xs, gate_proj, up_proj, down_proj, w_col)


def _sc_combine_body(ys_hbm, pab_hbm, out_hbm, idx_v, buf_a, buf_b, sem):
    wid = lax.axis_index("s") * NC + lax.axis_index("c")
    base = wid * TOK_W
    pltpu.sync_copy(pab_hbm.at[wid], idx_v)
    cp_a = pltpu.async_copy(ys_hbm.at[idx_v.at[0]], buf_a, sem)
    cp_b = pltpu.async_copy(ys_hbm.at[idx_v.at[1]], buf_b, sem)
    cp_a.wait()
    cp_b.wait()

    def row(r, _):
        for c in range(D // 16):
            sl = pl.ds(c * 16, 16)
            buf_a[r, sl] = buf_a[r, sl] + buf_b[r, sl]
        return 0

    lax.fori_loop(0, TOK_W, row, 0)
    pltpu.sync_copy(buf_a, out_hbm.at[pl.ds(base, TOK_W)])


def _sc_combine(ys, pos_a, pos_b):
    pab = jnp.stack([pos_a.reshape(NW, TOK_W),
                     pos_b.reshape(NW, TOK_W)], axis=1)
    mesh = plsc.VectorSubcoreMesh(core_axis_name="c", subcore_axis_name="s")
    run = functools.partial(
        pl.kernel, mesh=mesh,
        out_type=jax.ShapeDtypeStruct((T, D), jnp.float32),
        scratch_types=[
            pltpu.VMEM((2, TOK_W), jnp.int32),
            pltpu.VMEM((TOK_W, D), jnp.float32),
            pltpu.VMEM((TOK_W, D), jnp.float32),
            pltpu.SemaphoreType.DMA,
        ],
    )(_sc_combine_body)
    return run(ys, pab)


def kernel(hidden_states, gate_w, gate_proj, up_proj, down_proj):
    B, S, _ = hidden_states.shape
    hs2d = hidden_states.reshape(-1, D)
    combine = _router(hs2d, gate_w)
    token_map, w_col, tile_expert, tile_valid, pos_a, pos_b = _dispatch(combine)
    xs = _sc_gather(hs2d, token_map)
    ys = _gmm(tile_expert, tile_valid, xs, w_col, gate_proj, up_proj, down_proj)
    out = _sc_combine(ys, pos_a, pos_b)
    return out.reshape(B, S, D)


# speedup vs baseline: 1.0093x; 1.0093x over previous
import functools

import jax
import jax.numpy as jnp
from jax import lax
from jax.experimental import pallas as pl
from jax.experimental.pallas import tpu as pltpu
from jax.experimental.pallas import tpu_sc as plsc

E = 8
K = 2
D = 768
F = 2048
T = 2048

BT = 256
P = K * T + E * BT
N_TILES = P // BT

NC = 2
NS = 16
NW = NC * NS

ROWS_W = P // NW
GCHUNK = ROWS_W // 3
NCHUNK = ROWS_W // GCHUNK
TOK_W = T // NW


def _router_body(x_ref, gw_ref, sel_ref, rw_ref):
    x = x_ref[...]
    logits = lax.dot_general(x, gw_ref[...],
                             (((1,), (1,)), ((), ())),
                             preferred_element_type=jnp.float32)
    m = jnp.max(logits, axis=1, keepdims=True)
    ex = jnp.exp(logits - m)
    p = ex / jnp.sum(ex, axis=1, keepdims=True)
    col = lax.broadcasted_iota(jnp.int32, p.shape, 1)
    a1 = jnp.argmax(p, axis=1)[:, None]
    m1 = col == a1
    a2 = jnp.argmax(jnp.where(m1, -1.0, p), axis=1)[:, None]
    m2 = col == a2
    sel_ref[...] = jnp.concatenate([a1, a2], axis=1)
    w1 = jnp.max(jnp.where(m1, p, 0.0), axis=1, keepdims=True)
    w2 = jnp.max(jnp.where(m2, p, 0.0), axis=1, keepdims=True)
    rw_ref[...] = jnp.concatenate([w1, w2], axis=1)


def _router(hs2d, gate_w):
    bt = 1024
    return pl.pallas_call(
        _router_body,
        grid=(T // bt,),
        in_specs=[
            pl.BlockSpec((bt, D), lambda i: (i, 0)),
            pl.BlockSpec((E, D), lambda i: (0, 0)),
        ],
        out_specs=[
            pl.BlockSpec((bt, K), lambda i: (i, 0)),
            pl.BlockSpec((bt, K), lambda i: (i, 0)),
        ],
        out_shape=[
            jax.ShapeDtypeStruct((T, K), jnp.int32),
            jax.ShapeDtypeStruct((T, K), jnp.float32),
        ],
    )(hs2d, gate_w)


def _dispatch(sel, rw_top):
    e_flat = sel.reshape(-1)
    w_flat = rw_top.reshape(-1)
    onehot = (e_flat[:, None] == jnp.arange(E, dtype=jnp.int32)[None, :])
    counts = jnp.sum(onehot.astype(jnp.int32), axis=0)
    rank = jnp.sum((jnp.cumsum(onehot.astype(jnp.int32), axis=0) - 1)
                   * onehot.astype(jnp.int32), axis=1)
    pad_cnt = ((counts + BT - 1) // BT) * BT
    ends = jnp.cumsum(pad_cnt)
    pad_off = ends - pad_cnt
    pos = pad_off[e_flat] + rank

    tok = (jnp.arange(K * T, dtype=jnp.int32) // K)
    token_map = jnp.zeros((P,), jnp.int32).at[pos].set(tok)
    w_col = jnp.zeros((P,), jnp.float32).at[pos].set(w_flat).reshape(P, 1)

    total_pad = ends[-1]
    n_active = total_pad // BT
    row_start = jnp.arange(N_TILES, dtype=jnp.int32) * BT
    owner_raw = jnp.sum((row_start[:, None] >= ends[None, :]).astype(jnp.int32),
                        axis=1)
    tile_valid = (row_start < total_pad).astype(jnp.int32)
    last_owner = jnp.minimum(owner_raw, E - 1)[jnp.maximum(n_active - 1, 0)]
    tile_expert = jnp.where(tile_valid == 1,
                            jnp.minimum(owner_raw, E - 1), last_owner)
    tile_expert = tile_expert.astype(jnp.int32)

    pos_a = pos[0::K]
    pos_b = pos[1::K]
    return token_map, w_col, tile_expert, tile_valid, pos_a, pos_b


def _sc_gather_body(hs_hbm, tm_hbm, xs_hbm, idx_v, buf_a, buf_b, sem_a, sem_b):
    wid = lax.axis_index("s") * NC + lax.axis_index("c")
    base = wid * ROWS_W
    pltpu.sync_copy(tm_hbm.at[wid], idx_v)
    g0 = pltpu.async_copy(hs_hbm.at[idx_v.at[0]], buf_a, sem_a)
    g1 = pltpu.async_copy(hs_hbm.at[idx_v.at[1]], buf_b, sem_b)
    g0.wait()
    w0 = pltpu.async_copy(buf_a, xs_hbm.at[pl.ds(base, GCHUNK)], sem_a)
    g1.wait()
    w1 = pltpu.async_copy(buf_b, xs_hbm.at[pl.ds(base + GCHUNK, GCHUNK)], sem_b)
    w0.wait()
    g2 = pltpu.async_copy(hs_hbm.at[idx_v.at[2]], buf_a, sem_a)
    g2.wait()
    w2 = pltpu.async_copy(buf_a, xs_hbm.at[pl.ds(base + 2 * GCHUNK, GCHUNK)],
                          sem_a)
    w1.wait()
    w2.wait()


def _sc_gather(hs2d, token_map):
    tm = token_map.reshape(NW, NCHUNK, GCHUNK)
    mesh = plsc.VectorSubcoreMesh(core_axis_name="c", subcore_axis_name="s")
    run = functools.partial(
        pl.kernel, mesh=mesh,
        out_type=jax.ShapeDtypeStruct((P, D), jnp.float32),
        scratch_types=[
            pltpu.VMEM((NCHUNK, GCHUNK), jnp.int32),
            pltpu.VMEM((GCHUNK, D), jnp.float32),
            pltpu.VMEM((GCHUNK, D), jnp.float32),
            pltpu.SemaphoreType.DMA,
            pltpu.SemaphoreType.DMA,
        ],
    )(_sc_gather_body)
    return run(hs2d, tm)


def _gmm_body(te_ref, tv_ref, x_ref, g_ref, u_ref, d_ref, w_ref, y_ref):
    i = pl.program_id(0)

    @pl.when(tv_ref[i] != 0)
    def _():
        x = x_ref[...]
        g = lax.dot_general(x, g_ref[0], (((1,), (1,)), ((), ())),
                            preferred_element_type=jnp.float32)
        u = lax.dot_general(x, u_ref[0], (((1,), (1,)), ((), ())),
                            preferred_element_type=jnp.float32)
        act = g * jax.nn.sigmoid(g) * u
        y = lax.dot_general(act, d_ref[0], (((1,), (1,)), ((), ())),
                            preferred_element_type=jnp.float32)
        y_ref[...] = y * w_ref[...]


def _gmm(tile_expert, tile_valid, xs, w_col, gate_proj, up_proj, down_proj):
    grid_spec = pltpu.PrefetchScalarGridSpec(
        num_scalar_prefetch=2,
        grid=(N_TILES,),
        in_specs=[
            pl.BlockSpec((BT, D), lambda i, te, tv: (i, 0)),
            pl.BlockSpec((1, F, D), lambda i, te, tv: (te[i], 0, 0)),
            pl.BlockSpec((1, F, D), lambda i, te, tv: (te[i], 0, 0)),
            pl.BlockSpec((1, D, F), lambda i, te, tv: (te[i], 0, 0)),
            pl.BlockSpec((BT, 1), lambda i, te, tv: (i, 0)),
        ],
        out_specs=pl.BlockSpec((BT, D), lambda i, te, tv: (i, 0)),
    )
    return pl.pallas_call(
        _gmm_body,
        grid_spec=grid_spec,
        out_shape=jax.ShapeDtypeStruct((P, D), jnp.float32),
        compiler_params=pltpu.CompilerParams(
            dimension_semantics=("arbitrary",),
        ),
    )(tile_expert, tile_valid, xs, gate_proj, up_proj, down_proj, w_col)


def _sc_combine_body(ys_hbm, pab_hbm, out_hbm, idx_v, buf_a, buf_b, sem):
    wid = lax.axis_index("s") * NC + lax.axis_index("c")
    base = wid * TOK_W
    pltpu.sync_copy(pab_hbm.at[wid], idx_v)
    cp_a = pltpu.async_copy(ys_hbm.at[idx_v.at[0]], buf_a, sem)
    cp_b = pltpu.async_copy(ys_hbm.at[idx_v.at[1]], buf_b, sem)
    cp_a.wait()
    cp_b.wait()

    def row(r, _):
        for c in range(D // 16):
            sl = pl.ds(c * 16, 16)
            buf_a[r, sl] = buf_a[r, sl] + buf_b[r, sl]
        return 0

    lax.fori_loop(0, TOK_W, row, 0)
    pltpu.sync_copy(buf_a, out_hbm.at[pl.ds(base, TOK_W)])


def _sc_combine(ys, pos_a, pos_b):
    pab = jnp.stack([pos_a.reshape(NW, TOK_W),
                     pos_b.reshape(NW, TOK_W)], axis=1)
    mesh = plsc.VectorSubcoreMesh(core_axis_name="c", subcore_axis_name="s")
    run = functools.partial(
        pl.kernel, mesh=mesh,
        out_type=jax.ShapeDtypeStruct((T, D), jnp.float32),
        scratch_types=[
            pltpu.VMEM((2, TOK_W), jnp.int32),
            pltpu.VMEM((TOK_W, D), jnp.float32),
            pltpu.VMEM((TOK_W, D), jnp.float32),
            pltpu.SemaphoreType.DMA,
        ],
    )(_sc_combine_body)
    return run(ys, pab)


def kernel(hidden_states, gate_w, gate_proj, up_proj, down_proj):
    B, S, _ = hidden_states.shape
    hs2d = hidden_states.reshape(-1, D)
    sel, rw_top = _router(hs2d, gate_w)
    token_map, w_col, tile_expert, tile_valid, pos_a, pos_b = _dispatch(sel, rw_top)
    xs = _sc_gather(hs2d, token_map)
    ys = _gmm(tile_expert, tile_valid, xs, w_col, gate_proj, up_proj, down_proj)
    out = _sc_combine(ys, pos_a, pos_b)
    return out.reshape(B, S, D)
